# baseline (speedup 1.0000x reference)
import functools

import jax
import jax.numpy as jnp
from jax import lax
from jax.experimental import pallas as pl
from jax.experimental.pallas import tpu as pltpu
from jax.experimental.pallas import tpu_sc as plsc

VOCAB = 100000
EMBED = 64
BATCH = 1024
CTX = 20

_NC, _NS, _L = 2, 16, 16
_NW = _NC * _NS
_B_PER_W = BATCH // _NW
_IDX_PER_W = _B_PER_W * CTX


@functools.lru_cache(maxsize=1)
def _make_gather_sum():
    mesh = plsc.VectorSubcoreMesh(
        core_axis_name="c", subcore_axis_name="s", num_cores=_NC, num_subcores=_NS
    )

    @functools.partial(
        pl.kernel,
        mesh=mesh,
        out_type=jax.ShapeDtypeStruct((BATCH, EMBED), jnp.float32),
        scratch_types=[
            pltpu.VMEM((_IDX_PER_W,), jnp.int32),
            pltpu.VMEM((_IDX_PER_W, EMBED), jnp.float32),
            pltpu.VMEM((_B_PER_W, EMBED), jnp.float32),
            pltpu.SemaphoreType.DMA,
        ],
        compiler_params=pltpu.CompilerParams(use_tc_tiling_on_sc=False),
    )
    def gather_sum(idx_hbm, table_hbm, out_hbm, idx_v, rows_v, acc_v, sem):
        wid = lax.axis_index("s") * _NC + lax.axis_index("c")
        base = wid * _IDX_PER_W
        pltpu.sync_copy(idx_hbm.at[pl.ds(base, _IDX_PER_W)], idx_v)
        pltpu.async_copy(table_hbm.at[idx_v], rows_v, sem).wait()

        def body(b, carry):
            rbase = b * CTX
            for d in range(EMBED // _L):
                acc = rows_v[rbase, pl.ds(d * _L, _L)]
                for c in range(1, CTX):
                    acc = acc + rows_v[rbase + c, pl.ds(d * _L, _L)]
                acc_v[b, pl.ds(d * _L, _L)] = acc
            return carry

        lax.fori_loop(0, _B_PER_W, body, 0)
        pltpu.sync_copy(acc_v, out_hbm.at[pl.ds(wid * _B_PER_W, _B_PER_W)])

    return gather_sum


def _mm_body(x_ref, w_ref, b_ref, o_ref):
    o_ref[...] = (
        lax.dot_general(
            x_ref[...],
            w_ref[...],
            dimension_numbers=(((1,), (1,)), ((), ())),
            preferred_element_type=jnp.float32,
        )
        + b_ref[...]
    )


_BM = 8


def _fc1(x_sum, fc1_w, fc1_b):
    grid = (BATCH // _BM,)
    return pl.pallas_call(
        _mm_body,
        grid=grid,
        in_specs=[
            pl.BlockSpec((_BM, EMBED), lambda i: (i, 0)),
            pl.BlockSpec((VOCAB, EMBED), lambda i: (0, 0)),
            pl.BlockSpec((1, VOCAB), lambda i: (0, 0)),
        ],
        out_specs=pl.BlockSpec((_BM, VOCAB), lambda i: (i, 0)),
        out_shape=jax.ShapeDtypeStruct((BATCH, VOCAB), jnp.float32),
        compiler_params=pltpu.CompilerParams(
            dimension_semantics=("arbitrary",),
        ),
    )(x_sum, fc1_w, fc1_b.reshape(1, VOCAB))


def kernel(x_in, embedding, fc1_w, fc1_b):
    idx_flat = x_in.reshape(-1).astype(jnp.int32)
    x_sum = _make_gather_sum()(idx_flat, embedding)
    return _fc1(x_sum, fc1_w, fc1_b)


# speedup vs baseline: 2.1872x; 2.1872x over previous
import functools

import jax
import jax.numpy as jnp
from jax import lax
from jax.experimental import pallas as pl
from jax.experimental.pallas import tpu as pltpu
from jax.experimental.pallas import tpu_sc as plsc

VOCAB = 100000
EMBED = 64
BATCH = 1024
CTX = 20

_NC, _NS, _L = 2, 16, 16
_NW = _NC * _NS
_B_PER_W = BATCH // _NW
_IDX_PER_W = _B_PER_W * CTX


@functools.lru_cache(maxsize=1)
def _make_gather_sum():
    mesh = plsc.VectorSubcoreMesh(
        core_axis_name="c", subcore_axis_name="s", num_cores=_NC, num_subcores=_NS
    )

    @functools.partial(
        pl.kernel,
        mesh=mesh,
        out_type=jax.ShapeDtypeStruct((BATCH, EMBED), jnp.float32),
        scratch_types=[
            pltpu.VMEM((_IDX_PER_W,), jnp.int32),
            pltpu.VMEM((_IDX_PER_W, EMBED), jnp.float32),
            pltpu.VMEM((_B_PER_W, EMBED), jnp.float32),
            pltpu.SemaphoreType.DMA,
        ],
        compiler_params=pltpu.CompilerParams(use_tc_tiling_on_sc=False),
    )
    def gather_sum(idx_hbm, table_hbm, out_hbm, idx_v, rows_v, acc_v, sem):
        wid = lax.axis_index("s") * _NC + lax.axis_index("c")
        base = wid * _IDX_PER_W
        pltpu.sync_copy(idx_hbm.at[pl.ds(base, _IDX_PER_W)], idx_v)
        pltpu.async_copy(table_hbm.at[idx_v], rows_v, sem).wait()

        def body(b, carry):
            rbase = b * CTX
            for d in range(EMBED // _L):
                acc = rows_v[rbase, pl.ds(d * _L, _L)]
                for c in range(1, CTX):
                    acc = acc + rows_v[rbase + c, pl.ds(d * _L, _L)]
                acc_v[b, pl.ds(d * _L, _L)] = acc
            return carry

        lax.fori_loop(0, _B_PER_W, body, 0)
        pltpu.sync_copy(acc_v, out_hbm.at[pl.ds(wid * _B_PER_W, _B_PER_W)])

    return gather_sum


def _mm_body(x_ref, w_ref, b_ref, o_ref):
    o_ref[...] = (
        lax.dot_general(
            x_ref[...],
            w_ref[...],
            dimension_numbers=(((1,), (1,)), ((), ())),
            preferred_element_type=jnp.float32,
        )
        + b_ref[...]
    )


_BN = 2048


def _fc1(x_sum, fc1_w, fc1_b):
    grid = (pl.cdiv(VOCAB, _BN),)
    return pl.pallas_call(
        _mm_body,
        grid=grid,
        in_specs=[
            pl.BlockSpec((BATCH, EMBED), lambda j: (0, 0)),
            pl.BlockSpec((_BN, EMBED), lambda j: (j, 0)),
            pl.BlockSpec((1, _BN), lambda j: (0, j)),
        ],
        out_specs=pl.BlockSpec((BATCH, _BN), lambda j: (0, j)),
        out_shape=jax.ShapeDtypeStruct((BATCH, VOCAB), jnp.float32),
        compiler_params=pltpu.CompilerParams(
            dimension_semantics=("arbitrary",),
        ),
    )(x_sum, fc1_w, fc1_b.reshape(1, VOCAB))


def kernel(x_in, embedding, fc1_w, fc1_b):
    idx_flat = x_in.reshape(-1).astype(jnp.int32)
    x_sum = _make_gather_sum()(idx_flat, embedding)
    return _fc1(x_sum, fc1_w, fc1_b)


# speedup vs baseline: 6.0067x; 2.7463x over previous
import functools

import jax
import jax.numpy as jnp
from jax import lax
from jax.experimental import pallas as pl
from jax.experimental.pallas import tpu as pltpu
from jax.experimental.pallas import tpu_sc as plsc

VOCAB = 100000
EMBED = 64
BATCH = 1024
CTX = 20

_NC, _NS, _L = 2, 16, 16
_NW = _NC * _NS
_B_PER_W = BATCH // _NW
_IDX_PER_W = _B_PER_W * CTX


@functools.lru_cache(maxsize=1)
def _make_gather_sum():
    mesh = plsc.VectorSubcoreMesh(
        core_axis_name="c", subcore_axis_name="s", num_cores=_NC, num_subcores=_NS
    )

    @functools.partial(
        pl.kernel,
        mesh=mesh,
        out_type=jax.ShapeDtypeStruct((BATCH, EMBED), jnp.float32),
        scratch_types=[
            pltpu.VMEM((_IDX_PER_W,), jnp.int32),
            pltpu.VMEM((_IDX_PER_W, EMBED), jnp.float32),
            pltpu.VMEM((_B_PER_W, EMBED), jnp.float32),
            pltpu.SemaphoreType.DMA,
        ],
        compiler_params=pltpu.CompilerParams(use_tc_tiling_on_sc=False),
    )
    def gather_sum(idx_hbm, table_hbm, out_hbm, idx_v, rows_v, acc_v, sem):
        wid = lax.axis_index("s") * _NC + lax.axis_index("c")
        base = wid * _IDX_PER_W
        pltpu.sync_copy(idx_hbm.at[pl.ds(base, _IDX_PER_W)], idx_v)
        pltpu.async_copy(table_hbm.at[idx_v], rows_v, sem).wait()

        def body(b, carry):
            rbase = b * CTX
            for d in range(EMBED // _L):
                acc = rows_v[rbase, pl.ds(d * _L, _L)]
                for c in range(1, CTX):
                    acc = acc + rows_v[rbase + c, pl.ds(d * _L, _L)]
                acc_v[b, pl.ds(d * _L, _L)] = acc
            return carry

        lax.fori_loop(0, _B_PER_W, body, 0)
        pltpu.sync_copy(acc_v, out_hbm.at[pl.ds(wid * _B_PER_W, _B_PER_W)])

    return gather_sum


def _mm_body(wt_ref, x_ref, b_ref, o_ref):
    o_ref[...] = lax.dot_general(
        wt_ref[...],
        x_ref[...],
        dimension_numbers=(((0,), (1,)), ((), ())),
        preferred_element_type=jnp.float32,
    ) + jnp.transpose(b_ref[...])


_BV = 2048


def _fc1(x_sum, fc1_w, fc1_b):
    grid = (pl.cdiv(VOCAB, _BV),)
    out_t = pl.pallas_call(
        _mm_body,
        grid=grid,
        in_specs=[
            pl.BlockSpec((EMBED, _BV), lambda j: (0, j)),
            pl.BlockSpec((BATCH, EMBED), lambda j: (0, 0)),
            pl.BlockSpec((1, _BV), lambda j: (0, j)),
        ],
        out_specs=pl.BlockSpec((_BV, BATCH), lambda j: (j, 0)),
        out_shape=jax.ShapeDtypeStruct((VOCAB, BATCH), jnp.float32),
        compiler_params=pltpu.CompilerParams(
            dimension_semantics=("arbitrary",),
        ),
    )(fc1_w.T, x_sum, fc1_b.reshape(1, VOCAB))
    return out_t.T


def kernel(x_in, embedding, fc1_w, fc1_b):
    idx_flat = x_in.reshape(-1).astype(jnp.int32)
    x_sum = _make_gather_sum()(idx_flat, embedding)
    return _fc1(x_sum, fc1_w, fc1_b)


# speedup vs baseline: 6.0653x; 1.0098x over previous
import functools

import jax
import jax.numpy as jnp
from jax import lax
from jax.experimental import pallas as pl
from jax.experimental.pallas import tpu as pltpu
from jax.experimental.pallas import tpu_sc as plsc

VOCAB = 100000
EMBED = 64
BATCH = 1024
CTX = 20

_NC, _NS, _L = 2, 16, 16
_NW = _NC * _NS
_B_PER_W = BATCH // _NW
_IDX_PER_W = _B_PER_W * CTX


@functools.lru_cache(maxsize=1)
def _make_gather_sum():
    mesh = plsc.VectorSubcoreMesh(
        core_axis_name="c", subcore_axis_name="s", num_cores=_NC, num_subcores=_NS
    )

    @functools.partial(
        pl.kernel,
        mesh=mesh,
        out_type=jax.ShapeDtypeStruct((BATCH, EMBED), jnp.float32),
        scratch_types=[
            pltpu.VMEM((_IDX_PER_W,), jnp.int32),
            pltpu.VMEM((_IDX_PER_W, EMBED), jnp.float32),
            pltpu.VMEM((_B_PER_W, EMBED), jnp.float32),
            pltpu.SemaphoreType.DMA,
        ],
        compiler_params=pltpu.CompilerParams(use_tc_tiling_on_sc=False),
    )
    def gather_sum(idx_hbm, table_hbm, out_hbm, idx_v, rows_v, acc_v, sem):
        wid = lax.axis_index("s") * _NC + lax.axis_index("c")
        base = wid * _IDX_PER_W
        pltpu.sync_copy(idx_hbm.at[pl.ds(base, _IDX_PER_W)], idx_v)
        pltpu.async_copy(table_hbm.at[idx_v], rows_v, sem).wait()

        def body(b, carry):
            rbase = b * CTX
            for d in range(EMBED // _L):
                acc = rows_v[rbase, pl.ds(d * _L, _L)]
                for c in range(1, CTX):
                    acc = acc + rows_v[rbase + c, pl.ds(d * _L, _L)]
                acc_v[b, pl.ds(d * _L, _L)] = acc
            return carry

        lax.fori_loop(0, _B_PER_W, body, 0)
        pltpu.sync_copy(acc_v, out_hbm.at[pl.ds(wid * _B_PER_W, _B_PER_W)])

    return gather_sum


def _mm_body(wt_ref, x_ref, b_ref, o_ref):
    o_ref[...] = lax.dot_general(
        wt_ref[...],
        x_ref[...],
        dimension_numbers=(((0,), (1,)), ((), ())),
        preferred_element_type=jnp.float32,
    ) + jnp.transpose(b_ref[...])


_BV = 4096


def _fc1(x_sum, fc1_w, fc1_b):
    grid = (pl.cdiv(VOCAB, _BV),)
    out_t = pl.pallas_call(
        _mm_body,
        grid=grid,
        in_specs=[
            pl.BlockSpec((EMBED, _BV), lambda j: (0, j)),
            pl.BlockSpec((BATCH, EMBED), lambda j: (0, 0)),
            pl.BlockSpec((1, _BV), lambda j: (0, j)),
        ],
        out_specs=pl.BlockSpec((_BV, BATCH), lambda j: (j, 0)),
        out_shape=jax.ShapeDtypeStruct((VOCAB, BATCH), jnp.float32),
        compiler_params=pltpu.CompilerParams(
            dimension_semantics=("arbitrary",),
        ),
    )(fc1_w.T, x_sum, fc1_b.reshape(1, VOCAB))
    return out_t.T


def kernel(x_in, embedding, fc1_w, fc1_b):
    idx_flat = x_in.reshape(-1).astype(jnp.int32)
    x_sum = _make_gather_sum()(idx_flat, embedding)
    return _fc1(x_sum, fc1_w, fc1_b)
